# Initial kernel scaffold; baseline (speedup 1.0000x reference)
import jax
import jax.numpy as jnp
from jax.experimental import pallas as pl


def kernel(raw_seqs, remap, table):
    raise NotImplementedError("write your pallas kernel here")


# speedup vs baseline: 11.5069x; 11.5069x over previous
import functools

import jax
import jax.numpy as jnp
from jax import lax
from jax.experimental import pallas as pl
from jax.experimental.pallas import tpu as pltpu
from jax.experimental.pallas import tpu_sc as plsc

_NUM_CORES = 2
_NUM_SUBCORES = 16
_NUM_WORKERS = _NUM_CORES * _NUM_SUBCORES
_CHUNK = 128


@functools.partial(jax.jit, static_argnames=())
def _flat_lookup(flat_idx, remap, table):
    n = flat_idx.shape[0]
    t, d = table.shape
    per_w = n // _NUM_WORKERS
    n_chunks = per_w // _CHUNK
    assert per_w * _NUM_WORKERS == n and n_chunks * _CHUNK == per_w

    mesh = plsc.VectorSubcoreMesh(core_axis_name="c", subcore_axis_name="s")

    @functools.partial(
        pl.kernel,
        out_type=jax.ShapeDtypeStruct((n, d), jnp.float32),
        mesh=mesh,
        compiler_params=pltpu.CompilerParams(use_tc_tiling_on_sc=False),
        scratch_types=[
            pltpu.VMEM((_CHUNK,), jnp.int32),
            pltpu.VMEM((_CHUNK,), jnp.int32),
            pltpu.VMEM((_CHUNK, d), jnp.float32),
            pltpu.SemaphoreType.DMA,
        ],
    )
    def run(idx_hbm, remap_hbm, table_hbm, out_hbm, idx_v, emb_v, rows_v, sem):
        wid = lax.axis_index("s") * _NUM_CORES + lax.axis_index("c")
        base = wid * per_w

        def body(g, carry):
            off = base + g * _CHUNK
            pltpu.sync_copy(idx_hbm.at[pl.ds(off, _CHUNK)], idx_v)
            pltpu.async_copy(remap_hbm.at[idx_v], emb_v, sem).wait()
            pltpu.async_copy(table_hbm.at[emb_v], rows_v, sem).wait()
            pltpu.sync_copy(rows_v, out_hbm.at[pl.ds(off, _CHUNK)])
            return carry

        lax.fori_loop(0, n_chunks, body, 0)

    return run(flat_idx, remap, table)


def kernel(raw_seqs, remap, table):
    b, s = raw_seqs.shape
    d = table.shape[1]
    out = _flat_lookup(raw_seqs.reshape(b * s), remap, table)
    return out.reshape(b, s, d)


# speedup vs baseline: 16.9747x; 1.4752x over previous
import functools

import jax
import jax.numpy as jnp
from jax import lax
from jax.experimental import pallas as pl
from jax.experimental.pallas import tpu as pltpu
from jax.experimental.pallas import tpu_sc as plsc

_NUM_CORES = 2
_NUM_SUBCORES = 16
_NUM_WORKERS = _NUM_CORES * _NUM_SUBCORES
_CHUNK = 512
_NBUF = 2


def _flat_lookup(flat_idx, remap, table):
    n = flat_idx.shape[0]
    d = table.shape[1]
    per_w = n // _NUM_WORKERS
    n_chunks = per_w // _CHUNK
    assert per_w * _NUM_WORKERS == n and n_chunks * _CHUNK == per_w
    assert n_chunks % _NBUF == 0

    mesh = plsc.VectorSubcoreMesh(core_axis_name="c", subcore_axis_name="s")

    @functools.partial(
        pl.kernel,
        out_type=jax.ShapeDtypeStruct((n, d), jnp.float32),
        mesh=mesh,
        compiler_params=pltpu.CompilerParams(use_tc_tiling_on_sc=False),
        scratch_types=[
            pltpu.VMEM((per_w,), jnp.int32),
            pltpu.VMEM((per_w,), jnp.int32),
            pltpu.VMEM((_NBUF, _CHUNK, d), jnp.float32),
            pltpu.SemaphoreType.DMA,
            pltpu.SemaphoreType.DMA((_NBUF,)),
        ],
    )
    def run(idx_hbm, remap_hbm, table_hbm, out_hbm, idx_v, emb_v, rows_v,
            sem_in, sem_g):
        wid = lax.axis_index("s") * _NUM_CORES + lax.axis_index("c")
        base = wid * per_w

        pltpu.sync_copy(idx_hbm.at[pl.ds(base, per_w)], idx_v)
        pltpu.async_copy(remap_hbm.at[idx_v], emb_v, sem_in).wait()

        def gd(g, b):
            return pltpu.make_async_copy(
                table_hbm.at[emb_v.at[pl.ds(g * _CHUNK, _CHUNK)]],
                rows_v.at[b], sem_g.at[b])

        for b in range(_NBUF):
            gd(b, b).start()

        def body(i, carry):
            g0 = i * _NBUF
            for b in range(_NBUF):
                g = g0 + b
                gd(g, b).wait()
                pltpu.sync_copy(rows_v.at[b],
                                out_hbm.at[pl.ds(base + g * _CHUNK, _CHUNK)])
                gd(g + _NBUF, b).start()
            return carry

        lax.fori_loop(0, n_chunks // _NBUF - 1, body, 0)

        g0 = n_chunks - _NBUF
        for b in range(_NBUF):
            g = g0 + b
            gd(g, b).wait()
            pltpu.sync_copy(rows_v.at[b],
                            out_hbm.at[pl.ds(base + g * _CHUNK, _CHUNK)])

    return run(flat_idx, remap, table)


def kernel(raw_seqs, remap, table):
    b, s = raw_seqs.shape
    d = table.shape[1]
    out = _flat_lookup(raw_seqs.reshape(b * s), remap, table)
    return out.reshape(b, s, d)


# speedup vs baseline: 16.9760x; 1.0001x over previous
import functools

import jax
import jax.numpy as jnp
from jax import lax
from jax.experimental import pallas as pl
from jax.experimental.pallas import tpu as pltpu
from jax.experimental.pallas import tpu_sc as plsc

_NUM_CORES = 2
_NUM_SUBCORES = 16
_NUM_WORKERS = _NUM_CORES * _NUM_SUBCORES
_CHUNK = 256
_NBUF = 4
_LEAD = 2


def _flat_lookup(flat_idx, remap, table):
    n = flat_idx.shape[0]
    d = table.shape[1]
    per_w = n // _NUM_WORKERS
    n_chunks = per_w // _CHUNK
    assert per_w * _NUM_WORKERS == n and n_chunks * _CHUNK == per_w
    main_lo, main_hi = _NBUF - _LEAD, n_chunks - _LEAD
    assert (main_hi - main_lo) % _NBUF == 0

    mesh = plsc.VectorSubcoreMesh(core_axis_name="c", subcore_axis_name="s")

    @functools.partial(
        pl.kernel,
        out_type=jax.ShapeDtypeStruct((n, d), jnp.float32),
        mesh=mesh,
        compiler_params=pltpu.CompilerParams(use_tc_tiling_on_sc=False),
        scratch_types=[
            pltpu.VMEM((per_w,), jnp.int32),
            pltpu.VMEM((per_w,), jnp.int32),
            pltpu.VMEM((_NBUF, _CHUNK, d), jnp.float32),
            pltpu.SemaphoreType.DMA,
            pltpu.SemaphoreType.DMA((_NBUF,)),
            pltpu.SemaphoreType.DMA((_NBUF,)),
        ],
    )
    def run(idx_hbm, remap_hbm, table_hbm, out_hbm, idx_v, emb_v, rows_v,
            sem_in, sem_g, sem_s):
        wid = lax.axis_index("s") * _NUM_CORES + lax.axis_index("c")
        base = wid * per_w

        pltpu.sync_copy(idx_hbm.at[pl.ds(base, per_w)], idx_v)
        pltpu.async_copy(remap_hbm.at[idx_v], emb_v, sem_in).wait()

        def gd(g, b):
            return pltpu.make_async_copy(
                table_hbm.at[emb_v.at[pl.ds(g * _CHUNK, _CHUNK)]],
                rows_v.at[b], sem_g.at[b])

        def sd(g, b):
            return pltpu.make_async_copy(
                rows_v.at[b], out_hbm.at[pl.ds(base + g * _CHUNK, _CHUNK)],
                sem_s.at[b])

        for g in range(_LEAD):
            gd(g, g % _NBUF).start()

        def visit(g, b, store_wait, restart):
            gd(g, b).wait()
            sd(g, b).start()
            if restart:
                bn = (g + _LEAD) % _NBUF
                if store_wait:
                    sd(g + _LEAD - _NBUF, bn).wait()
                gd(g + _LEAD, bn).start()
            elif store_wait:
                bn = (g + _LEAD) % _NBUF
                sd(g + _LEAD - _NBUF, bn).wait()

        for g in range(main_lo):
            visit(g, g % _NBUF, store_wait=False, restart=True)

        def body(i, carry):
            g0 = main_lo + i * _NBUF
            for j in range(_NBUF):
                visit(g0 + j, (main_lo + j) % _NBUF, store_wait=True,
                      restart=True)
            return carry

        lax.fori_loop(0, (main_hi - main_lo) // _NBUF, body, 0)

        for g in range(main_hi, n_chunks):
            visit(g, g % _NBUF, store_wait=True, restart=False)

        for g in range(main_hi, n_chunks):
            sd(g, g % _NBUF).wait()

    return run(flat_idx, remap, table)


def kernel(raw_seqs, remap, table):
    b, s = raw_seqs.shape
    d = table.shape[1]
    out = _flat_lookup(raw_seqs.reshape(b * s), remap, table)
    return out.reshape(b, s, d)
